# baseline (speedup 1.0000x reference)
import functools

import jax
import jax.numpy as jnp
import numpy as np
from jax import lax
from jax.experimental import pallas as pl
from jax.experimental.pallas import tpu as pltpu
from jax.experimental.pallas import tpu_sc as plsc

_FIELD_DIMS = [1000] * 26
_F = len(_FIELD_DIMS)
_V = sum(_FIELD_DIMS)
_D = 32
_B = 4096
_OFFS = np.array((0, *np.cumsum(_FIELD_DIMS)[:-1]), dtype=np.int32)
_I, _J = np.triu_indices(_F, k=1)
_NPAIR = _I.size
_NROW = 2 * _NPAIR
_NROW_PAD = 656
_NTILE = 32
_BPT = _B // _NTILE
_CH = 32
_NCH = _BPT // _CH


_CHUNKS = [(c * 128, 128 if c < 5 else _NROW_PAD - 5 * 128) for c in range(6)]


def _ffm_body(table, ipairs, idxp, wlin, blin, out,
              ip_v, ix_v, rows0, rows1, wlin_v, blin_v, out_v, sem0, sem1):
    nc = 2
    wid = lax.axis_index("s") * nc + lax.axis_index("c")
    base = wid * _BPT

    pltpu.sync_copy(wlin, wlin_v)
    pltpu.sync_copy(blin, blin_v)
    b0vec = blin_v[pl.ds(0, 16)]
    lane = lax.iota(jnp.int32, 16)

    def fire(lb, rows_v, sem):
        for off, n in _CHUNKS:
            pltpu.async_copy(table.at[ip_v.at[lb, pl.ds(off, n)]],
                             rows_v.at[pl.ds(off, n)], sem)

    def drain(rows_v, sem):
        for off, n in _CHUNKS:
            pltpu.make_async_copy(table.at[ip_v.at[0, pl.ds(off, n)]],
                                  rows_v.at[pl.ds(off, n)], sem).wait()

    def compute(lb, rows_v, ch, res):
        def pbody(i, acc):
            for u in range(5):
                p = i * 5 + u
                l1 = rows_v[2 * p, pl.ds(0, 16)]
                r1 = rows_v[2 * p + 1, pl.ds(0, 16)]
                l2 = rows_v[2 * p, pl.ds(16, 16)]
                r2 = rows_v[2 * p + 1, pl.ds(16, 16)]
                acc = acc + l1 * r1 + l2 * r2
            return acc

        acc = lax.fori_loop(0, _NPAIR // 5, pbody,
                            jnp.zeros((16,), jnp.float32))
        g1 = plsc.load_gather(wlin_v, [ix_v[lb, pl.ds(0, 16)]])
        g2 = plsc.load_gather(wlin_v, [ix_v[lb, pl.ds(16, 16)]])
        tot = acc + g1 + g2
        s = jnp.sum(tot) + b0vec[0]
        res = jnp.where(lane == (lb & 15), s, res)

        @pl.when((lb & 15) == 15)
        def _():
            out_v[pl.ds(ch * _CH + lb - 15, 16)] = res

        return res

    for ch in range(_NCH):
        b0 = base + ch * _CH
        pltpu.sync_copy(ipairs.at[pl.ds(b0, _CH)], ip_v)
        pltpu.sync_copy(idxp.at[pl.ds(b0, _CH)], ix_v)
        fire(0, rows0, sem0)

        def body2(t, res):
            lb0 = 2 * t
            lb1 = lb0 + 1
            fire(lb1, rows1, sem1)
            drain(rows0, sem0)
            res = compute(lb0, rows0, ch, res)

            @pl.when(lb1 < _CH - 1)
            def _():
                fire(lb0 + 2, rows0, sem0)

            drain(rows1, sem1)
            res = compute(lb1, rows1, ch, res)
            return res

        lax.fori_loop(0, _CH // 2, body2, jnp.zeros((16,), jnp.float32))

    pltpu.sync_copy(out_v, out.at[pl.ds(base, _BPT)])


@jax.jit
def kernel(x, W_lin, b_lin, W_emb):
    offs = jnp.asarray(_OFFS)
    idx = x + offs[None, :]
    li = idx[:, _I] + jnp.asarray(_J * _V, dtype=jnp.int32)
    ri = idx[:, _J] + jnp.asarray(_I * _V, dtype=jnp.int32)
    inter = jnp.stack([li, ri], axis=2).reshape(_B, _NROW)
    ipairs = jnp.concatenate(
        [inter, jnp.zeros((_B, _NROW_PAD - _NROW), jnp.int32)], axis=1)
    idxp = jnp.concatenate(
        [idx, jnp.full((_B, 32 - _F), _V, jnp.int32)], axis=1)
    table = W_emb.reshape(_F * _V, _D)
    wlin_pad = jnp.concatenate([W_lin[:, 0], jnp.zeros((8,), jnp.float32)])
    blin_pad = jnp.concatenate([b_lin, jnp.zeros((15,), jnp.float32)])

    mesh = plsc.VectorSubcoreMesh(core_axis_name="c", subcore_axis_name="s")
    run = functools.partial(
        pl.kernel, _ffm_body,
        out_type=jax.ShapeDtypeStruct((_B,), jnp.float32),
        mesh=mesh,
        compiler_params=pltpu.CompilerParams(
            needs_layout_passes=False, use_tc_tiling_on_sc=False),
        scratch_types=[
            pltpu.VMEM((_CH, _NROW_PAD), jnp.int32),
            pltpu.VMEM((_CH, 32), jnp.int32),
            pltpu.VMEM((_NROW_PAD, _D), jnp.float32),
            pltpu.VMEM((_NROW_PAD, _D), jnp.float32),
            pltpu.VMEM((_V + 8,), jnp.float32),
            pltpu.VMEM((16,), jnp.float32),
            pltpu.VMEM((_BPT,), jnp.float32),
            pltpu.SemaphoreType.DMA,
            pltpu.SemaphoreType.DMA,
        ],
    )()
    return run(table, ipairs, idxp, wlin_pad, blin_pad)


# speedup vs baseline: 1.0995x; 1.0995x over previous
import functools

import jax
import jax.numpy as jnp
import numpy as np
from jax import lax
from jax.experimental import pallas as pl
from jax.experimental.pallas import tpu as pltpu
from jax.experimental.pallas import tpu_sc as plsc

_FIELD_DIMS = [1000] * 26
_F = len(_FIELD_DIMS)
_V = sum(_FIELD_DIMS)
_D = 32
_B = 4096
_OFFS = np.array((0, *np.cumsum(_FIELD_DIMS)[:-1]), dtype=np.int32)
_I, _J = np.triu_indices(_F, k=1)
_NPAIR = _I.size
_NROW = 2 * _NPAIR
_NROW_PAD = 656
_NTILE = 32
_BPT = _B // _NTILE
_CH = 32
_NCH = _BPT // _CH


_CHUNKS = [(c * 128, 128 if c < 5 else _NROW_PAD - 5 * 128) for c in range(6)]


def _ffm_body(table, ipairs, idxp, wlin, blin, out,
              ip_v, ix_v, rows0, rows1, wlin_v, blin_v, out_v, sem0, sem1):
    nc = 2
    wid = lax.axis_index("s") * nc + lax.axis_index("c")
    base = wid * _BPT

    pltpu.sync_copy(wlin, wlin_v)
    pltpu.sync_copy(blin, blin_v)
    b0vec = blin_v[pl.ds(0, 16)]
    lane = lax.iota(jnp.int32, 16)

    def fire(lb, rows_v, sem):
        for off, n in _CHUNKS:
            pltpu.async_copy(table.at[ip_v.at[lb, pl.ds(off, n)]],
                             rows_v.at[pl.ds(off, n)], sem)

    def drain(rows_v, sem):
        for off, n in _CHUNKS:
            pltpu.make_async_copy(table.at[ip_v.at[0, pl.ds(off, n)]],
                                  rows_v.at[pl.ds(off, n)], sem).wait()

    def compute(lb, rows_v, ch, res):
        def pbody(i, acc):
            for u in range(5):
                p = i * 5 + u
                lo, hi = plsc.unpack(rows_v[2 * p, pl.ds(0, _D)],
                                     format=plsc.PackFormat.INTERLEAVED)
                ro, rh = plsc.unpack(rows_v[2 * p + 1, pl.ds(0, _D)],
                                     format=plsc.PackFormat.INTERLEAVED)
                acc = acc + lo * ro + hi * rh
            return acc

        acc = lax.fori_loop(0, _NPAIR // 5, pbody,
                            jnp.zeros((16,), jnp.float32))
        g1 = plsc.load_gather(wlin_v, [ix_v[lb, pl.ds(0, 16)]])
        g2 = plsc.load_gather(wlin_v, [ix_v[lb, pl.ds(16, 16)]])
        tot = acc + g1 + g2
        s = jnp.sum(tot) + b0vec[0]
        res = jnp.where(lane == (lb & 15), s, res)

        @pl.when((lb & 15) == 15)
        def _():
            out_v[pl.ds(ch * _CH + lb - 15, 16)] = res

        return res

    for ch in range(_NCH):
        b0 = base + ch * _CH
        pltpu.sync_copy(ipairs.at[pl.ds(b0, _CH)], ip_v)
        pltpu.sync_copy(idxp.at[pl.ds(b0, _CH)], ix_v)
        fire(0, rows0, sem0)

        def body2(t, res):
            lb0 = 2 * t
            lb1 = lb0 + 1
            fire(lb1, rows1, sem1)
            drain(rows0, sem0)
            res = compute(lb0, rows0, ch, res)

            @pl.when(lb1 < _CH - 1)
            def _():
                fire(lb0 + 2, rows0, sem0)

            drain(rows1, sem1)
            res = compute(lb1, rows1, ch, res)
            return res

        lax.fori_loop(0, _CH // 2, body2, jnp.zeros((16,), jnp.float32))

    pltpu.sync_copy(out_v, out.at[pl.ds(base, _BPT)])


@jax.jit
def kernel(x, W_lin, b_lin, W_emb):
    offs = jnp.asarray(_OFFS)
    idx = x + offs[None, :]
    li = idx[:, _I] + jnp.asarray(_J * _V, dtype=jnp.int32)
    ri = idx[:, _J] + jnp.asarray(_I * _V, dtype=jnp.int32)
    inter = jnp.stack([li, ri], axis=2).reshape(_B, _NROW)
    ipairs = jnp.concatenate(
        [inter, jnp.zeros((_B, _NROW_PAD - _NROW), jnp.int32)], axis=1)
    idxp = jnp.concatenate(
        [idx, jnp.full((_B, 32 - _F), _V, jnp.int32)], axis=1)
    table = W_emb.astype(jnp.bfloat16).reshape(_F * _V, _D)
    wlin_pad = jnp.concatenate([W_lin[:, 0], jnp.zeros((8,), jnp.float32)])
    blin_pad = jnp.concatenate([b_lin, jnp.zeros((15,), jnp.float32)])

    mesh = plsc.VectorSubcoreMesh(core_axis_name="c", subcore_axis_name="s")
    run = functools.partial(
        pl.kernel, _ffm_body,
        out_type=jax.ShapeDtypeStruct((_B,), jnp.float32),
        mesh=mesh,
        compiler_params=pltpu.CompilerParams(
            needs_layout_passes=False, use_tc_tiling_on_sc=False),
        scratch_types=[
            pltpu.VMEM((_CH, _NROW_PAD), jnp.int32),
            pltpu.VMEM((_CH, 32), jnp.int32),
            pltpu.VMEM((_NROW_PAD, _D), jnp.bfloat16),
            pltpu.VMEM((_NROW_PAD, _D), jnp.bfloat16),
            pltpu.VMEM((_V + 8,), jnp.float32),
            pltpu.VMEM((16,), jnp.float32),
            pltpu.VMEM((_BPT,), jnp.float32),
            pltpu.SemaphoreType.DMA,
            pltpu.SemaphoreType.DMA,
        ],
    )()
    return run(table, ipairs, idxp, wlin_pad, blin_pad)


# speedup vs baseline: 1.1558x; 1.0512x over previous
import functools

import jax
import jax.numpy as jnp
import numpy as np
from jax import lax
from jax.experimental import pallas as pl
from jax.experimental.pallas import tpu as pltpu
from jax.experimental.pallas import tpu_sc as plsc

_FIELD_DIMS = [1000] * 26
_F = len(_FIELD_DIMS)
_V = sum(_FIELD_DIMS)
_D = 32
_B = 4096
_OFFS = np.array((0, *np.cumsum(_FIELD_DIMS)[:-1]), dtype=np.int32)
_FP = 32
_NTILE = 32
_BPT = _B // _NTILE
_CH = 32
_NCH = _BPT // _CH
_NPAD = _FP - _F


def _ffm_body(wemb, idxp, wlin, blin, out,
              ix_v, rows0, rows1, wlin_v, blin_v, out_v, sem0, sem1):
    nc = 2
    wid = lax.axis_index("s") * nc + lax.axis_index("c")
    base = wid * _BPT

    pltpu.sync_copy(wlin, wlin_v)
    pltpu.sync_copy(blin, blin_v)
    b0vec = blin_v[pl.ds(0, 16)]
    w0vec = wlin_v[pl.ds(0, 16)]
    lane = lax.iota(jnp.int32, 16)

    def fire(lb, rows_v, sem):
        def tbody(ft, _):
            pltpu.async_copy(wemb.at[ft].at[ix_v.at[lb]],
                             rows_v.at[pl.ds(ft * _FP, _FP)], sem)
            return 0

        lax.fori_loop(0, _F, tbody, 0)

    def drain(rows_v, sem):
        def tbody(ft, _):
            pltpu.make_async_copy(wemb.at[0].at[ix_v.at[0]],
                                  rows_v.at[pl.ds(ft * _FP, _FP)], sem).wait()
            return 0

        lax.fori_loop(0, _F, tbody, 0)

    def compute(lb, rows_v, ch, res):
        def ibody(i, acc):
            def jbody(j, acc2):
                lo, hi = plsc.unpack(rows_v[j * _FP + i, pl.ds(0, _D)],
                                     format=plsc.PackFormat.INTERLEAVED)
                ro, rh = plsc.unpack(rows_v[i * _FP + j, pl.ds(0, _D)],
                                     format=plsc.PackFormat.INTERLEAVED)
                return acc2 + lo * ro + hi * rh

            return lax.fori_loop(i + 1, _F, jbody, acc)

        acc = lax.fori_loop(0, _F - 1, ibody, jnp.zeros((16,), jnp.float32))
        g1 = plsc.load_gather(wlin_v, [ix_v[lb, pl.ds(0, 16)]])
        g2 = plsc.load_gather(wlin_v, [ix_v[lb, pl.ds(16, 16)]])
        s = (jnp.sum(acc) + jnp.sum(g1 + g2)
             - jnp.float32(_NPAD) * w0vec[0] + b0vec[0])
        res = jnp.where(lane == (lb & 15), s, res)

        @pl.when((lb & 15) == 15)
        def _():
            out_v[pl.ds(ch * _CH + lb - 15, 16)] = res

        return res

    for ch in range(_NCH):
        b0 = base + ch * _CH
        pltpu.sync_copy(idxp.at[pl.ds(b0, _CH)], ix_v)
        fire(0, rows0, sem0)

        def body2(t, res):
            lb0 = 2 * t
            lb1 = lb0 + 1
            fire(lb1, rows1, sem1)
            drain(rows0, sem0)
            res = compute(lb0, rows0, ch, res)

            @pl.when(lb1 < _CH - 1)
            def _():
                fire(lb0 + 2, rows0, sem0)

            drain(rows1, sem1)
            res = compute(lb1, rows1, ch, res)
            return res

        lax.fori_loop(0, _CH // 2, body2, jnp.zeros((16,), jnp.float32))

    pltpu.sync_copy(out_v, out.at[pl.ds(base, _BPT)])


@jax.jit
def kernel(x, W_lin, b_lin, W_emb):
    offs = jnp.asarray(_OFFS)
    idx = x + offs[None, :]
    idxp = jnp.concatenate(
        [idx, jnp.zeros((_B, _FP - _F), jnp.int32)], axis=1)
    table = W_emb.astype(jnp.bfloat16)
    wlin_pad = jnp.concatenate([W_lin[:, 0], jnp.zeros((8,), jnp.float32)])
    blin_pad = jnp.concatenate([b_lin, jnp.zeros((15,), jnp.float32)])

    mesh = plsc.VectorSubcoreMesh(core_axis_name="c", subcore_axis_name="s")
    run = functools.partial(
        pl.kernel, _ffm_body,
        out_type=jax.ShapeDtypeStruct((_B,), jnp.float32),
        mesh=mesh,
        compiler_params=pltpu.CompilerParams(
            needs_layout_passes=False, use_tc_tiling_on_sc=False),
        scratch_types=[
            pltpu.VMEM((_CH, _FP), jnp.int32),
            pltpu.VMEM((_F * _FP, _D), jnp.bfloat16),
            pltpu.VMEM((_F * _FP, _D), jnp.bfloat16),
            pltpu.VMEM((_V + 8,), jnp.float32),
            pltpu.VMEM((16,), jnp.float32),
            pltpu.VMEM((_BPT,), jnp.float32),
            pltpu.SemaphoreType.DMA,
            pltpu.SemaphoreType.DMA,
        ],
    )()
    return run(table, idxp, wlin_pad, blin_pad)
